# baseline (speedup 1.0000x reference)
import functools

import jax
import jax.numpy as jnp
from jax import lax
from jax.experimental import pallas as pl
from jax.experimental.pallas import tpu as pltpu
from jax.experimental.pallas import tpu_sc as plsc

N = 10000
E = 320000
FD = 128
DT = 0.1
STEPS = 100

NSC = 16
NCH = 640
NP = NCH * 16
SL = NP // NSC
EPT = E // NSC


def _diffusion_body(row_hbm, col_hbm, dve_hbm, dvd_hbm,
                    out_hbm, part_hbm, sfin_hbm,
                    row_v, col_v, dve_v, s_v, acc_v, dvd_v, rbuf_v, own_v, sem_d):
    cid = lax.axis_index("c")
    sid = lax.axis_index("s")
    ebase = sid * EPT
    pltpu.sync_copy(row_hbm.at[pl.ds(ebase, EPT)], row_v)
    pltpu.sync_copy(col_hbm.at[pl.ds(ebase, EPT)], col_v)
    pltpu.sync_copy(dve_hbm.at[pl.ds(ebase, EPT)], dve_v)
    pltpu.sync_copy(dvd_hbm, dvd_v)

    zeros16 = jnp.zeros((16,), jnp.float32)
    ones16 = jnp.ones((16,), jnp.float32)
    lanes = lax.iota(jnp.int32, 16)
    sbase = sid * SL

    def init_body(c, carry):
        s_v[pl.ds(c * 16, 16)] = ones16
        return carry

    lax.fori_loop(0, NCH, init_body, 0)

    def step(it, carry):
        par = lax.rem(it, 2)

        def zacc(c, c2):
            for u in range(4):
                acc_v[pl.ds(c * 64 + u * 16, 16)] = zeros16
            return c2

        lax.fori_loop(0, NCH // 4, zacc, 0)

        def edge(k, c2):
            for u in range(5):
                off = k * 80 + u * 16
                idx = col_v[pl.ds(off, 16)]
                g = plsc.load_gather(s_v, [idx])
                contrib = g * dve_v[pl.ds(off, 16)]
                r = row_v[pl.ds(off, 16)]
                plsc.addupdate_scatter(acc_v, [r], contrib)
            return c2

        lax.fori_loop(0, EPT // 80, edge, 0)
        pltpu.sync_copy(acc_v, part_hbm.at[par, cid, sid])
        plsc.subcore_barrier()

        descs = [
            pltpu.async_copy(
                part_hbm.at[par, cid, j, pl.ds(sbase, SL)], rbuf_v.at[j], sem_d)
            for j in range(NSC)
        ]
        for d in descs:
            d.wait()

        def red(j, c2):
            off = j * 16
            tot = dvd_v[pl.ds(sbase + off, 16)] * s_v[pl.ds(sbase + off, 16)]

            def radd(k, t):
                return t + rbuf_v[k, pl.ds(off, 16)]

            tot = lax.fori_loop(0, NSC, radd, tot)
            own_v[pl.ds(off, 16)] = tot
            return c2

        lax.fori_loop(0, SL // 16, red, 0)
        pltpu.sync_copy(own_v, sfin_hbm.at[par, cid, pl.ds(sbase, SL)])
        plsc.subcore_barrier()
        pltpu.sync_copy(sfin_hbm.at[par, cid], s_v)
        return carry

    lax.fori_loop(0, STEPS, step, 0)

    @pl.when((cid == 0) & (sid == 0))
    def _finalize():
        def mx(c, m):
            a = jnp.abs(s_v[pl.ds(c * 16, 16)])
            a = jnp.where(c == 0, jnp.where(lanes > 0, a, zeros16), a)
            return jnp.maximum(m, a)

        m_vec = lax.fori_loop(0, NCH, mx, zeros16)
        m = jnp.max(m_vec)
        scale = ones16 / jnp.maximum(jnp.broadcast_to(m, (16,)), 1e-12)

        def wr(c, c2):
            o = s_v[pl.ds(c * 16, 16)] * scale
            o = jnp.where(c == 0, jnp.where(lanes == 0, ones16, o), o)
            acc_v[pl.ds(c * 16, 16)] = o
            return c2

        lax.fori_loop(0, NCH, wr, 0)
        pltpu.sync_copy(acc_v, out_hbm)


_diffusion_kernel = functools.partial(
    pl.kernel,
    out_type=(
        jax.ShapeDtypeStruct((NP,), jnp.float32),
        jax.ShapeDtypeStruct((2, 2, NSC, NP), jnp.float32),
        jax.ShapeDtypeStruct((2, 2, NP), jnp.float32),
    ),
    mesh=plsc.VectorSubcoreMesh(core_axis_name="c", subcore_axis_name="s"),
    compiler_params=pltpu.CompilerParams(needs_layout_passes=False),
    scratch_types=[
        pltpu.VMEM((EPT,), jnp.int32),
        pltpu.VMEM((EPT,), jnp.int32),
        pltpu.VMEM((EPT,), jnp.float32),
        pltpu.VMEM((NP,), jnp.float32),
        pltpu.VMEM((NP,), jnp.float32),
        pltpu.VMEM((NP,), jnp.float32),
        pltpu.VMEM((NSC, SL), jnp.float32),
        pltpu.VMEM((SL,), jnp.float32),
        pltpu.SemaphoreType.DMA,
    ],
)(_diffusion_body)


NW = 32
EPW = E // NW
GB = 80


def _att_pair_body(g1t_hbm, g2t_hbm, row_hbm, col_hbm, b_hbm, out_hbm,
                   g1tbl_v, g2tbl_v, row_v, col_v, b_v, obuf_v):
    cid = lax.axis_index("c")
    sid = lax.axis_index("s")
    wid = sid * 2 + cid
    base = wid * EPW
    pltpu.sync_copy(row_hbm.at[pl.ds(base, EPW)], row_v)
    pltpu.sync_copy(col_hbm.at[pl.ds(base, EPW)], col_v)
    pltpu.sync_copy(b_hbm, b_v)

    def plane(k, carry):
        pltpu.sync_copy(g1t_hbm.at[pl.ds(k * NP, NP)], g1tbl_v)
        pltpu.sync_copy(g2t_hbm.at[pl.ds(k * NP, NP)], g2tbl_v)
        bk = plsc.load_gather(b_v, [jnp.broadcast_to(k, (16,)).astype(jnp.int32)])

        def chunk(c, c2):
            off = c * 16
            r = row_v[pl.ds(off, 16)]
            cc = col_v[pl.ds(off, 16)]
            x = plsc.load_gather(g1tbl_v, [r]) + plsc.load_gather(g2tbl_v, [cc]) + bk
            obuf_v[pl.ds(off, 16)] = jnp.where(x >= 0, x, 0.1 * x)
            return c2

        lax.fori_loop(0, EPW // 16, chunk, 0)
        pltpu.sync_copy(obuf_v, out_hbm.at[pl.ds(k * E + base, EPW)])
        return carry

    lax.fori_loop(0, 16, plane, 0)


_att_pair = functools.partial(
    pl.kernel,
    out_type=jax.ShapeDtypeStruct((16 * E,), jnp.float32),
    mesh=plsc.VectorSubcoreMesh(core_axis_name="c", subcore_axis_name="s"),
    compiler_params=pltpu.CompilerParams(needs_layout_passes=False),
    scratch_types=[
        pltpu.VMEM((NP,), jnp.float32),
        pltpu.VMEM((NP,), jnp.float32),
        pltpu.VMEM((EPW,), jnp.int32),
        pltpu.VMEM((EPW,), jnp.int32),
        pltpu.VMEM((16,), jnp.float32),
        pltpu.VMEM((EPW,), jnp.float32),
    ],
)(_att_pair_body)


def _prop_body(dinv_hbm, s_hbm, row_hbm, col_hbm, ev_hbm, prop_hbm, vnorm_hbm,
               dinv_v, s_v, row_v, col_v, ev_v, o1_v, o2_v):
    cid = lax.axis_index("c")
    sid = lax.axis_index("s")
    wid = sid * 2 + cid
    base = wid * EPW
    pltpu.sync_copy(dinv_hbm, dinv_v)
    pltpu.sync_copy(s_hbm, s_v)
    pltpu.sync_copy(row_hbm.at[pl.ds(base, EPW)], row_v)
    pltpu.sync_copy(col_hbm.at[pl.ds(base, EPW)], col_v)
    pltpu.sync_copy(ev_hbm.at[pl.ds(base, EPW)], ev_v)
    eps16 = jnp.full((16,), 1e-12, jnp.float32)

    def chunk(k, carry):
        off = k * 16
        r = row_v[pl.ds(off, 16)]
        c = col_v[pl.ds(off, 16)]
        v = ev_v[pl.ds(off, 16)]
        dr = plsc.load_gather(dinv_v, [r])
        dc = plsc.load_gather(dinv_v, [c])
        o1_v[pl.ds(off, 16)] = dr * v * dc
        sr = plsc.load_gather(s_v, [r])
        o2_v[pl.ds(off, 16)] = jnp.abs(v) / jnp.maximum(sr, eps16)
        return carry

    lax.fori_loop(0, EPW // 16, chunk, 0)
    pltpu.sync_copy(o1_v, prop_hbm.at[pl.ds(base, EPW)])
    pltpu.sync_copy(o2_v, vnorm_hbm.at[pl.ds(base, EPW)])


_prop_kernel = functools.partial(
    pl.kernel,
    out_type=(
        jax.ShapeDtypeStruct((E,), jnp.float32),
        jax.ShapeDtypeStruct((E,), jnp.float32),
    ),
    mesh=plsc.VectorSubcoreMesh(core_axis_name="c", subcore_axis_name="s"),
    compiler_params=pltpu.CompilerParams(needs_layout_passes=False),
    scratch_types=[
        pltpu.VMEM((NP,), jnp.float32),
        pltpu.VMEM((NP,), jnp.float32),
        pltpu.VMEM((EPW,), jnp.int32),
        pltpu.VMEM((EPW,), jnp.int32),
        pltpu.VMEM((EPW,), jnp.float32),
        pltpu.VMEM((EPW,), jnp.float32),
        pltpu.VMEM((EPW,), jnp.float32),
    ],
)(_prop_body)


def _dve_body(ssum_hbm, ex_hbm, row_hbm, col_hbm, ev_hbm, dve_hbm,
              ssum_v, ex_v, row_v, col_v, ev_v, o_v):
    cid = lax.axis_index("c")
    sid = lax.axis_index("s")
    wid = sid * 2 + cid
    base = wid * EPW
    pltpu.sync_copy(ssum_hbm, ssum_v)
    pltpu.sync_copy(ex_hbm.at[pl.ds(base, EPW)], ex_v)
    pltpu.sync_copy(row_hbm.at[pl.ds(base, EPW)], row_v)
    pltpu.sync_copy(col_hbm.at[pl.ds(base, EPW)], col_v)
    pltpu.sync_copy(ev_hbm.at[pl.ds(base, EPW)], ev_v)
    eps16 = jnp.full((16,), 1e-12, jnp.float32)
    zeros16 = jnp.zeros((16,), jnp.float32)
    ones16 = jnp.ones((16,), jnp.float32)
    dt16 = jnp.full((16,), DT, jnp.float32)

    def chunk(k, carry):
        off = k * 16
        r = row_v[pl.ds(off, 16)]
        c = col_v[pl.ds(off, 16)]
        sr = plsc.load_gather(ssum_v, [r])
        a = ex_v[pl.ds(off, 16)] / jnp.maximum(sr, eps16)
        pol = -jnp.sign(ev_v[pl.ds(off, 16)]) * a
        dve = dt16 * pol
        dve = jnp.where(r == 0, jnp.where(c == 0, ones16, zeros16), dve)
        o_v[pl.ds(off, 16)] = dve
        return carry

    lax.fori_loop(0, EPW // 16, chunk, 0)
    pltpu.sync_copy(o_v, dve_hbm.at[pl.ds(base, EPW)])


_dve_kernel = functools.partial(
    pl.kernel,
    out_type=jax.ShapeDtypeStruct((E,), jnp.float32),
    mesh=plsc.VectorSubcoreMesh(core_axis_name="c", subcore_axis_name="s"),
    compiler_params=pltpu.CompilerParams(needs_layout_passes=False),
    scratch_types=[
        pltpu.VMEM((NP,), jnp.float32),
        pltpu.VMEM((EPW,), jnp.float32),
        pltpu.VMEM((EPW,), jnp.int32),
        pltpu.VMEM((EPW,), jnp.int32),
        pltpu.VMEM((EPW,), jnp.float32),
        pltpu.VMEM((EPW,), jnp.float32),
    ],
)(_dve_body)


EB = 6400
PC = 4


def _gcn_planar_body(D):
    EPC = E // 2
    passes = (D // 16) // PC
    NBAT = EPC // EB

    def body(xwt_hbm, row_hbm, col_hbm, prop_hbm, out_hbm,
             rb0, cb0, pb0, rb1, cb1, pb1,
             t0, t1, t2, t3, a0, a1, a2, a3, sem0, sem1):
        cid = lax.axis_index("c")
        sid = lax.axis_index("s")
        ebase = cid * EPC
        zeros16 = jnp.zeros((16,), jnp.float32)
        tbls = [t0, t1, t2, t3]
        accs = [a0, a1, a2, a3]
        bufs = [(rb0, cb0, pb0, sem0), (rb1, cb1, pb1, sem1)]

        def fire(b, par):
            off = ebase + b * EB
            rb, cb, pb, sem = bufs[par]
            return [
                pltpu.async_copy(row_hbm.at[pl.ds(off, EB)], rb, sem),
                pltpu.async_copy(col_hbm.at[pl.ds(off, EB)], cb, sem),
                pltpu.async_copy(prop_hbm.at[pl.ds(off, EB)], pb, sem),
            ]

        def do_pass(ps, carry):
            pbase = sid * (D // 16) + ps * PC
            for j in range(PC):
                pltpu.sync_copy(xwt_hbm.at[pl.ds((pbase + j) * NP, NP)], tbls[j])

            def zacc(c, c2):
                for j in range(PC):
                    accs[j][pl.ds(c * 16, 16)] = zeros16
                return c2

            lax.fori_loop(0, NCH, zacc, 0)

            descs = fire(0, 0)
            for b in range(NBAT):
                par = b % 2
                nxt = fire(b + 1, 1 - par) if b + 1 < NBAT else None
                for d in descs:
                    d.wait()
                rb, cb, pb, _ = bufs[par]

                def chunk(k, c3):
                    for u in range(4):
                        o = k * 64 + u * 16
                        r = rb[pl.ds(o, 16)]
                        cc = cb[pl.ds(o, 16)]
                        pr = pb[pl.ds(o, 16)]
                        for j in range(PC):
                            g = plsc.load_gather(tbls[j], [r])
                            plsc.addupdate_scatter(accs[j], [cc], g * pr)
                    return c3

                lax.fori_loop(0, EB // 64, chunk, 0)
                descs = nxt
            for j in range(PC):
                pltpu.sync_copy(
                    accs[j], out_hbm.at[pl.ds((cid * D + pbase + j) * NP, NP)])
            return carry

        lax.fori_loop(0, passes, do_pass, 0)

    return body


def _make_gcn_planar(D):
    return functools.partial(
        pl.kernel,
        out_type=jax.ShapeDtypeStruct((2 * D * NP,), jnp.float32),
        mesh=plsc.VectorSubcoreMesh(core_axis_name="c", subcore_axis_name="s"),
        compiler_params=pltpu.CompilerParams(needs_layout_passes=False),
        scratch_types=[
            pltpu.VMEM((EB,), jnp.int32),
            pltpu.VMEM((EB,), jnp.int32),
            pltpu.VMEM((EB,), jnp.float32),
            pltpu.VMEM((EB,), jnp.int32),
            pltpu.VMEM((EB,), jnp.int32),
            pltpu.VMEM((EB,), jnp.float32),
        ] + [pltpu.VMEM((NP,), jnp.float32)] * 8 + [
            pltpu.SemaphoreType.DMA,
            pltpu.SemaphoreType.DMA,
        ],
    )(_gcn_planar_body(D))


_gcn_planar_256 = _make_gcn_planar(256)
_gcn_planar_128 = _make_gcn_planar(128)


def _seg_scalar_body(idx_hbm, val_hbm, out_hbm, idx_v, val_v, acc_v):
    cid = lax.axis_index("c")
    sid = lax.axis_index("s")
    wid = sid * 2 + cid
    base = wid * EPW
    pltpu.sync_copy(idx_hbm.at[pl.ds(base, EPW)], idx_v)
    pltpu.sync_copy(val_hbm.at[pl.ds(base, EPW)], val_v)
    zeros16 = jnp.zeros((16,), jnp.float32)

    def zacc(c, c2):
        acc_v[pl.ds(c * 16, 16)] = zeros16
        return c2

    lax.fori_loop(0, NCH, zacc, 0)

    def chunk(k, c2):
        o = k * 16
        plsc.addupdate_scatter(acc_v, [idx_v[pl.ds(o, 16)]], val_v[pl.ds(o, 16)])
        return c2

    lax.fori_loop(0, EPW // 16, chunk, 0)
    pltpu.sync_copy(acc_v, out_hbm.at[pl.ds(wid * NP, NP)])


_seg_scalar = functools.partial(
    pl.kernel,
    out_type=jax.ShapeDtypeStruct((NW * NP,), jnp.float32),
    mesh=plsc.VectorSubcoreMesh(core_axis_name="c", subcore_axis_name="s"),
    compiler_params=pltpu.CompilerParams(needs_layout_passes=False),
    scratch_types=[
        pltpu.VMEM((EPW,), jnp.int32),
        pltpu.VMEM((EPW,), jnp.float32),
        pltpu.VMEM((NP,), jnp.float32),
    ],
)(_seg_scalar_body)


def _seg_abs_cnt_body(idx_hbm, ev_hbm, s_hbm, cnt_hbm, idx_v, ev_v, acc1_v, acc2_v):
    cid = lax.axis_index("c")
    sid = lax.axis_index("s")
    wid = sid * 2 + cid
    base = wid * EPW
    pltpu.sync_copy(idx_hbm.at[pl.ds(base, EPW)], idx_v)
    pltpu.sync_copy(ev_hbm.at[pl.ds(base, EPW)], ev_v)
    zeros16 = jnp.zeros((16,), jnp.float32)
    ones16 = jnp.ones((16,), jnp.float32)

    def zacc(c, c2):
        acc1_v[pl.ds(c * 16, 16)] = zeros16
        acc2_v[pl.ds(c * 16, 16)] = zeros16
        return c2

    lax.fori_loop(0, NCH, zacc, 0)

    def chunk(k, c2):
        o = k * 16
        r = idx_v[pl.ds(o, 16)]
        plsc.addupdate_scatter(acc1_v, [r], jnp.abs(ev_v[pl.ds(o, 16)]))
        plsc.addupdate_scatter(acc2_v, [r], ones16)
        return c2

    lax.fori_loop(0, EPW // 16, chunk, 0)
    pltpu.sync_copy(acc1_v, s_hbm.at[pl.ds(wid * NP, NP)])
    pltpu.sync_copy(acc2_v, cnt_hbm.at[pl.ds(wid * NP, NP)])


_seg_abs_cnt = functools.partial(
    pl.kernel,
    out_type=(
        jax.ShapeDtypeStruct((NW * NP,), jnp.float32),
        jax.ShapeDtypeStruct((NW * NP,), jnp.float32),
    ),
    mesh=plsc.VectorSubcoreMesh(core_axis_name="c", subcore_axis_name="s"),
    compiler_params=pltpu.CompilerParams(needs_layout_passes=False),
    scratch_types=[
        pltpu.VMEM((EPW,), jnp.int32),
        pltpu.VMEM((EPW,), jnp.float32),
        pltpu.VMEM((NP,), jnp.float32),
        pltpu.VMEM((NP,), jnp.float32),
    ],
)(_seg_abs_cnt_body)


def _emax_body(e_hbm, out_hbm, e_v, m_v):
    cid = lax.axis_index("c")
    sid = lax.axis_index("s")
    wid = sid * 2 + cid
    base = wid * EPW
    pltpu.sync_copy(e_hbm.at[pl.ds(base, EPW)], e_v)

    def mx(k, m):
        return jnp.maximum(m, e_v[pl.ds(k * 16, 16)])

    m = lax.fori_loop(1, EPW // 16, mx, e_v[pl.ds(0, 16)])
    m_v[pl.ds(0, 16)] = m
    pltpu.sync_copy(m_v, out_hbm.at[pl.ds(wid * 16, 16)])


_emax = functools.partial(
    pl.kernel,
    out_type=jax.ShapeDtypeStruct((NW * 16,), jnp.float32),
    mesh=plsc.VectorSubcoreMesh(core_axis_name="c", subcore_axis_name="s"),
    compiler_params=pltpu.CompilerParams(needs_layout_passes=False),
    scratch_types=[
        pltpu.VMEM((EPW,), jnp.float32),
        pltpu.VMEM((16,), jnp.float32),
    ],
)(_emax_body)


def _exp_body(e_hbm, m_hbm, out_hbm, e_v, m_v, o_v):
    cid = lax.axis_index("c")
    sid = lax.axis_index("s")
    wid = sid * 2 + cid
    base = wid * EPW
    pltpu.sync_copy(e_hbm.at[pl.ds(base, EPW)], e_v)
    pltpu.sync_copy(m_hbm, m_v)
    m16 = m_v[pl.ds(0, 16)]

    def chunk(k, c2):
        o = k * 16
        o_v[pl.ds(o, 16)] = jnp.exp(e_v[pl.ds(o, 16)] - m16)
        return c2

    lax.fori_loop(0, EPW // 16, chunk, 0)
    pltpu.sync_copy(o_v, out_hbm.at[pl.ds(base, EPW)])


_exp_kernel = functools.partial(
    pl.kernel,
    out_type=jax.ShapeDtypeStruct((E,), jnp.float32),
    mesh=plsc.VectorSubcoreMesh(core_axis_name="c", subcore_axis_name="s"),
    compiler_params=pltpu.CompilerParams(needs_layout_passes=False),
    scratch_types=[
        pltpu.VMEM((EPW,), jnp.float32),
        pltpu.VMEM((16,), jnp.float32),
        pltpu.VMEM((EPW,), jnp.float32),
    ],
)(_exp_body)


def _leaky(x):
    return jnp.where(x >= 0, x, 0.1 * x)


NB = 512
EBK = 6400


def _dense1_body(x1_ref, w1_ref, b1_ref, w2_ref, lb2_ref, out_ref):
    x = w1_ref[...] * x1_ref[...] + b1_ref[...]
    x = _leaky(x)
    out_ref[...] = lax.dot_general(
        w2_ref[...], x, (((0,), (0,)), ((), ())),
        preferred_element_type=jnp.float32) + lb2_ref[...]


_dense1 = pl.pallas_call(
    _dense1_body,
    grid=(NP // NB,),
    in_specs=[
        pl.BlockSpec((1, NB), lambda i: (0, i)),
        pl.BlockSpec((FD, 1), lambda i: (0, 0)),
        pl.BlockSpec((FD, 1), lambda i: (0, 0)),
        pl.BlockSpec((FD, 2 * FD), lambda i: (0, 0)),
        pl.BlockSpec((2 * FD, 1), lambda i: (0, 0)),
    ],
    out_specs=pl.BlockSpec((2 * FD, NB), lambda i: (0, i)),
    out_shape=jax.ShapeDtypeStruct((2 * FD, NP), jnp.float32),
)


def _dense2_body(g0_ref, g1_ref, pd_ref, xw_ref, b_ref, w_ref, lb_ref, out_ref):
    x = _leaky(g0_ref[...] + g1_ref[...] + pd_ref[...] * xw_ref[...] + b_ref[...])
    out_ref[...] = lax.dot_general(
        w_ref[...], x, (((0,), (0,)), ((), ())),
        preferred_element_type=jnp.float32) + lb_ref[...]


def _make_dense2(Din, Dout):
    return pl.pallas_call(
        _dense2_body,
        grid=(NP // NB,),
        in_specs=[
            pl.BlockSpec((Din, NB), lambda i: (0, i)),
            pl.BlockSpec((Din, NB), lambda i: (0, i)),
            pl.BlockSpec((1, NB), lambda i: (0, i)),
            pl.BlockSpec((Din, NB), lambda i: (0, i)),
            pl.BlockSpec((Din, 1), lambda i: (0, 0)),
            pl.BlockSpec((Din, Dout), lambda i: (0, 0)),
            pl.BlockSpec((Dout, 1), lambda i: (0, 0)),
        ],
        out_specs=pl.BlockSpec((Dout, NB), lambda i: (0, i)),
        out_shape=jax.ShapeDtypeStruct((Dout, NP), jnp.float32),
    )


_dense2_256_128 = _make_dense2(2 * FD, FD)
_dense2_128_32 = _make_dense2(FD, 32)


def _attmlp_body(xet_ref, vn_ref, a2t_ref, a2v_ref, a2b_ref, cw_ref, out_ref):
    x = xet_ref[...]
    y = jnp.dot(a2t_ref[...], x, preferred_element_type=jnp.float32)
    y = y + a2v_ref[...] * vn_ref[...] + a2b_ref[...]
    y = _leaky(y)
    out_ref[...] = jnp.dot(cw_ref[...], y, preferred_element_type=jnp.float32)


_attmlp = pl.pallas_call(
    _attmlp_body,
    grid=(E // EBK,),
    in_specs=[
        pl.BlockSpec((16, EBK), lambda i: (0, i)),
        pl.BlockSpec((1, EBK), lambda i: (0, i)),
        pl.BlockSpec((17, 16), lambda i: (0, 0)),
        pl.BlockSpec((17, 1), lambda i: (0, 0)),
        pl.BlockSpec((17, 1), lambda i: (0, 0)),
        pl.BlockSpec((1, 17), lambda i: (0, 0)),
    ],
    out_specs=pl.BlockSpec((1, EBK), lambda i: (0, i)),
    out_shape=jax.ShapeDtypeStruct((1, E), jnp.float32),
)


def kernel(edge_index, edge_val, lin1_W, lin1_b, c2_W, c2_lb, c2_b, c3_W, c3_lb, c3_b, a1_W, a1_b, a2_W, a2_b, cW, cb, T):
    n = N
    row = edge_index[0]
    col = edge_index[1]
    ev = edge_val
    abs_val = jnp.abs(ev)

    def pad_n(v):
        return jnp.pad(v, (0, NP - n))

    sp, cp = _seg_abs_cnt(row, ev)
    s = sp.reshape(NW, NP).sum(0)[:n]
    cnt = cp.reshape(NW, NP).sum(0)[:n]
    abs_mean = s / jnp.maximum(cnt, 1.0)
    deg = s + jnp.abs(abs_mean)
    dinv = 1.0 / jnp.sqrt(jnp.maximum(deg, 1e-12))
    prop_e, vnorm = _prop_kernel(pad_n(dinv), pad_n(s), row, col, ev)
    prop_d = dinv * abs_mean * dinv

    x1 = _seg_scalar(col, prop_e).reshape(NW, NP).sum(0)[:n] + prop_d
    x1p = jnp.pad(x1, (0, NP - n))[None]
    pd_p = jnp.pad(prop_d, (0, NP - n))[None]
    xw2 = _dense1(x1p, lin1_W.T, lin1_b[:, None], c2_W, c2_lb[:, None])
    p2 = _gcn_planar_256(xw2.reshape(-1), row, col, prop_e).reshape(2, 2 * FD, NP)
    xw3 = _dense2_256_128(p2[0], p2[1], pd_p, xw2, c2_b[:, None], c3_W,
                          c3_lb[:, None])
    p3 = _gcn_planar_128(xw3.reshape(-1), row, col, prop_e).reshape(2, FD, NP)
    a1_cat = jnp.concatenate([a1_W[:FD], a1_W[FD:]], axis=1)
    g12 = _dense2_128_32(p3[0], p3[1], pd_p, xw3, c3_b[:, None], a1_cat,
                         jnp.zeros((32, 1), jnp.float32))

    g1t = g12[:16].reshape(-1)
    g2t = g12[16:].reshape(-1)
    xet = _att_pair(g1t, g2t, row, col, a1_b).reshape(16, E)
    e = _attmlp(xet, vnorm[None], a2_W[:-1].T, a2_W[-1][:, None],
                a2_b[:, None], cW.T)[0] + cb[0]
    M = jnp.max(_emax(e))
    ex = _exp_kernel(e, jnp.broadcast_to(M, (16,)))
    ssum = _seg_scalar(row, ex).reshape(NW, NP).sum(0)[:n]

    dve = _dve_kernel(pad_n(ssum), ex, row, col, ev)
    nid = jnp.arange(NP)
    dvd = jnp.where(nid < n, jnp.where(nid == 0, 1.0, 1.0 - DT), 0.0)
    dvd = dvd.astype(jnp.float32)
    spins, _, _ = _diffusion_kernel(row, col, dve, dvd)
    return spins[:n][:, None]


# speedup vs baseline: 1.0125x; 1.0125x over previous
import functools

import jax
import jax.numpy as jnp
from jax import lax
from jax.experimental import pallas as pl
from jax.experimental.pallas import tpu as pltpu
from jax.experimental.pallas import tpu_sc as plsc

N = 10000
E = 320000
FD = 128
DT = 0.1
STEPS = 100

NSC = 16
NCH = 640
NP = NCH * 16
SL = NP // NSC
EPT = E // NSC


def _diffusion_body(row_hbm, col_hbm, dve_hbm, dvd_hbm,
                    out_hbm, part_hbm, sfin_hbm,
                    row_v, col_v, dve_v, s_v, acc_v, dvd_v, rbuf_v, own_v, sem_d):
    cid = lax.axis_index("c")
    sid = lax.axis_index("s")
    ebase = sid * EPT
    pltpu.sync_copy(row_hbm.at[pl.ds(ebase, EPT)], row_v)
    pltpu.sync_copy(col_hbm.at[pl.ds(ebase, EPT)], col_v)
    pltpu.sync_copy(dve_hbm.at[pl.ds(ebase, EPT)], dve_v)
    pltpu.sync_copy(dvd_hbm, dvd_v)

    zeros16 = jnp.zeros((16,), jnp.float32)
    ones16 = jnp.ones((16,), jnp.float32)
    lanes = lax.iota(jnp.int32, 16)
    sbase = sid * SL

    def init_body(c, carry):
        s_v[pl.ds(c * 16, 16)] = ones16
        return carry

    lax.fori_loop(0, NCH, init_body, 0)

    def zacc(c, c2):
        for u in range(4):
            acc_v[pl.ds(c * 64 + u * 16, 16)] = zeros16
        return c2

    lax.fori_loop(0, NCH // 4, zacc, 0)

    def step(it, carry):
        par = lax.rem(it, 2)

        def edge(k, c2):
            for u in range(5):
                off = k * 80 + u * 16
                idx = col_v[pl.ds(off, 16)]
                g = plsc.load_gather(s_v, [idx])
                contrib = g * dve_v[pl.ds(off, 16)]
                r = row_v[pl.ds(off, 16)]
                plsc.addupdate_scatter(acc_v, [r], contrib)
            return c2

        lax.fori_loop(0, EPT // 80, edge, 0)
        pltpu.sync_copy(acc_v, part_hbm.at[par, cid, sid])
        plsc.subcore_barrier()

        descs = [
            pltpu.async_copy(
                part_hbm.at[par, cid, j, pl.ds(sbase, SL)], rbuf_v.at[j], sem_d)
            for j in range(NSC)
        ]
        for d in descs:
            d.wait()

        def red(j, c2):
            off = j * 16
            tot = dvd_v[pl.ds(sbase + off, 16)] * s_v[pl.ds(sbase + off, 16)]

            def radd(k, t):
                return t + rbuf_v[k, pl.ds(off, 16)]

            tot = lax.fori_loop(0, NSC, radd, tot)
            own_v[pl.ds(off, 16)] = tot
            return c2

        lax.fori_loop(0, SL // 16, red, 0)
        pltpu.sync_copy(own_v, sfin_hbm.at[par, cid, pl.ds(sbase, SL)])
        plsc.subcore_barrier()
        rb_d = pltpu.async_copy(sfin_hbm.at[par, cid], s_v, sem_d)
        lax.fori_loop(0, NCH // 4, zacc, 0)
        rb_d.wait()
        return carry

    lax.fori_loop(0, STEPS, step, 0)

    @pl.when((cid == 0) & (sid == 0))
    def _finalize():
        def mx(c, m):
            a = jnp.abs(s_v[pl.ds(c * 16, 16)])
            a = jnp.where(c == 0, jnp.where(lanes > 0, a, zeros16), a)
            return jnp.maximum(m, a)

        m_vec = lax.fori_loop(0, NCH, mx, zeros16)
        m = jnp.max(m_vec)
        scale = ones16 / jnp.maximum(jnp.broadcast_to(m, (16,)), 1e-12)

        def wr(c, c2):
            o = s_v[pl.ds(c * 16, 16)] * scale
            o = jnp.where(c == 0, jnp.where(lanes == 0, ones16, o), o)
            acc_v[pl.ds(c * 16, 16)] = o
            return c2

        lax.fori_loop(0, NCH, wr, 0)
        pltpu.sync_copy(acc_v, out_hbm)


_diffusion_kernel = functools.partial(
    pl.kernel,
    out_type=(
        jax.ShapeDtypeStruct((NP,), jnp.float32),
        jax.ShapeDtypeStruct((2, 2, NSC, NP), jnp.float32),
        jax.ShapeDtypeStruct((2, 2, NP), jnp.float32),
    ),
    mesh=plsc.VectorSubcoreMesh(core_axis_name="c", subcore_axis_name="s"),
    compiler_params=pltpu.CompilerParams(needs_layout_passes=False),
    scratch_types=[
        pltpu.VMEM((EPT,), jnp.int32),
        pltpu.VMEM((EPT,), jnp.int32),
        pltpu.VMEM((EPT,), jnp.float32),
        pltpu.VMEM((NP,), jnp.float32),
        pltpu.VMEM((NP,), jnp.float32),
        pltpu.VMEM((NP,), jnp.float32),
        pltpu.VMEM((NSC, SL), jnp.float32),
        pltpu.VMEM((SL,), jnp.float32),
        pltpu.SemaphoreType.DMA,
    ],
)(_diffusion_body)


NW = 32
EPW = E // NW
GB = 80


def _att_pair_body(g1t_hbm, g2t_hbm, row_hbm, col_hbm, b_hbm, out_hbm,
                   g1tbl_v, g2tbl_v, row_v, col_v, b_v, obuf_v):
    cid = lax.axis_index("c")
    sid = lax.axis_index("s")
    wid = sid * 2 + cid
    base = wid * EPW
    pltpu.sync_copy(row_hbm.at[pl.ds(base, EPW)], row_v)
    pltpu.sync_copy(col_hbm.at[pl.ds(base, EPW)], col_v)
    pltpu.sync_copy(b_hbm, b_v)

    def plane(k, carry):
        pltpu.sync_copy(g1t_hbm.at[pl.ds(k * NP, NP)], g1tbl_v)
        pltpu.sync_copy(g2t_hbm.at[pl.ds(k * NP, NP)], g2tbl_v)
        bk = plsc.load_gather(b_v, [jnp.broadcast_to(k, (16,)).astype(jnp.int32)])

        def chunk(c, c2):
            off = c * 16
            r = row_v[pl.ds(off, 16)]
            cc = col_v[pl.ds(off, 16)]
            x = plsc.load_gather(g1tbl_v, [r]) + plsc.load_gather(g2tbl_v, [cc]) + bk
            obuf_v[pl.ds(off, 16)] = jnp.where(x >= 0, x, 0.1 * x)
            return c2

        lax.fori_loop(0, EPW // 16, chunk, 0)
        pltpu.sync_copy(obuf_v, out_hbm.at[pl.ds(k * E + base, EPW)])
        return carry

    lax.fori_loop(0, 16, plane, 0)


_att_pair = functools.partial(
    pl.kernel,
    out_type=jax.ShapeDtypeStruct((16 * E,), jnp.float32),
    mesh=plsc.VectorSubcoreMesh(core_axis_name="c", subcore_axis_name="s"),
    compiler_params=pltpu.CompilerParams(needs_layout_passes=False),
    scratch_types=[
        pltpu.VMEM((NP,), jnp.float32),
        pltpu.VMEM((NP,), jnp.float32),
        pltpu.VMEM((EPW,), jnp.int32),
        pltpu.VMEM((EPW,), jnp.int32),
        pltpu.VMEM((16,), jnp.float32),
        pltpu.VMEM((EPW,), jnp.float32),
    ],
)(_att_pair_body)


def _prop_body(dinv_hbm, s_hbm, row_hbm, col_hbm, ev_hbm, prop_hbm, vnorm_hbm,
               dinv_v, s_v, row_v, col_v, ev_v, o1_v, o2_v):
    cid = lax.axis_index("c")
    sid = lax.axis_index("s")
    wid = sid * 2 + cid
    base = wid * EPW
    pltpu.sync_copy(dinv_hbm, dinv_v)
    pltpu.sync_copy(s_hbm, s_v)
    pltpu.sync_copy(row_hbm.at[pl.ds(base, EPW)], row_v)
    pltpu.sync_copy(col_hbm.at[pl.ds(base, EPW)], col_v)
    pltpu.sync_copy(ev_hbm.at[pl.ds(base, EPW)], ev_v)
    eps16 = jnp.full((16,), 1e-12, jnp.float32)

    def chunk(k, carry):
        off = k * 16
        r = row_v[pl.ds(off, 16)]
        c = col_v[pl.ds(off, 16)]
        v = ev_v[pl.ds(off, 16)]
        dr = plsc.load_gather(dinv_v, [r])
        dc = plsc.load_gather(dinv_v, [c])
        o1_v[pl.ds(off, 16)] = dr * v * dc
        sr = plsc.load_gather(s_v, [r])
        o2_v[pl.ds(off, 16)] = jnp.abs(v) / jnp.maximum(sr, eps16)
        return carry

    lax.fori_loop(0, EPW // 16, chunk, 0)
    pltpu.sync_copy(o1_v, prop_hbm.at[pl.ds(base, EPW)])
    pltpu.sync_copy(o2_v, vnorm_hbm.at[pl.ds(base, EPW)])


_prop_kernel = functools.partial(
    pl.kernel,
    out_type=(
        jax.ShapeDtypeStruct((E,), jnp.float32),
        jax.ShapeDtypeStruct((E,), jnp.float32),
    ),
    mesh=plsc.VectorSubcoreMesh(core_axis_name="c", subcore_axis_name="s"),
    compiler_params=pltpu.CompilerParams(needs_layout_passes=False),
    scratch_types=[
        pltpu.VMEM((NP,), jnp.float32),
        pltpu.VMEM((NP,), jnp.float32),
        pltpu.VMEM((EPW,), jnp.int32),
        pltpu.VMEM((EPW,), jnp.int32),
        pltpu.VMEM((EPW,), jnp.float32),
        pltpu.VMEM((EPW,), jnp.float32),
        pltpu.VMEM((EPW,), jnp.float32),
    ],
)(_prop_body)


def _dve_body(ssum_hbm, ex_hbm, row_hbm, col_hbm, ev_hbm, dve_hbm,
              ssum_v, ex_v, row_v, col_v, ev_v, o_v):
    cid = lax.axis_index("c")
    sid = lax.axis_index("s")
    wid = sid * 2 + cid
    base = wid * EPW
    pltpu.sync_copy(ssum_hbm, ssum_v)
    pltpu.sync_copy(ex_hbm.at[pl.ds(base, EPW)], ex_v)
    pltpu.sync_copy(row_hbm.at[pl.ds(base, EPW)], row_v)
    pltpu.sync_copy(col_hbm.at[pl.ds(base, EPW)], col_v)
    pltpu.sync_copy(ev_hbm.at[pl.ds(base, EPW)], ev_v)
    eps16 = jnp.full((16,), 1e-12, jnp.float32)
    zeros16 = jnp.zeros((16,), jnp.float32)
    ones16 = jnp.ones((16,), jnp.float32)
    dt16 = jnp.full((16,), DT, jnp.float32)

    def chunk(k, carry):
        off = k * 16
        r = row_v[pl.ds(off, 16)]
        c = col_v[pl.ds(off, 16)]
        sr = plsc.load_gather(ssum_v, [r])
        a = ex_v[pl.ds(off, 16)] / jnp.maximum(sr, eps16)
        pol = -jnp.sign(ev_v[pl.ds(off, 16)]) * a
        dve = dt16 * pol
        dve = jnp.where(r == 0, jnp.where(c == 0, ones16, zeros16), dve)
        o_v[pl.ds(off, 16)] = dve
        return carry

    lax.fori_loop(0, EPW // 16, chunk, 0)
    pltpu.sync_copy(o_v, dve_hbm.at[pl.ds(base, EPW)])


_dve_kernel = functools.partial(
    pl.kernel,
    out_type=jax.ShapeDtypeStruct((E,), jnp.float32),
    mesh=plsc.VectorSubcoreMesh(core_axis_name="c", subcore_axis_name="s"),
    compiler_params=pltpu.CompilerParams(needs_layout_passes=False),
    scratch_types=[
        pltpu.VMEM((NP,), jnp.float32),
        pltpu.VMEM((EPW,), jnp.float32),
        pltpu.VMEM((EPW,), jnp.int32),
        pltpu.VMEM((EPW,), jnp.int32),
        pltpu.VMEM((EPW,), jnp.float32),
        pltpu.VMEM((EPW,), jnp.float32),
    ],
)(_dve_body)


EB = 6400
PC = 4


def _gcn_planar_body(D):
    EPC = E // 2
    passes = (D // 16) // PC
    NBAT = EPC // EB

    def body(xwt_hbm, row_hbm, col_hbm, prop_hbm, out_hbm,
             rb0, cb0, pb0, rb1, cb1, pb1,
             t0, t1, t2, t3, a0, a1, a2, a3, sem0, sem1):
        cid = lax.axis_index("c")
        sid = lax.axis_index("s")
        ebase = cid * EPC
        zeros16 = jnp.zeros((16,), jnp.float32)
        tbls = [t0, t1, t2, t3]
        accs = [a0, a1, a2, a3]
        bufs = [(rb0, cb0, pb0, sem0), (rb1, cb1, pb1, sem1)]

        def fire(b, par):
            off = ebase + b * EB
            rb, cb, pb, sem = bufs[par]
            return [
                pltpu.async_copy(row_hbm.at[pl.ds(off, EB)], rb, sem),
                pltpu.async_copy(col_hbm.at[pl.ds(off, EB)], cb, sem),
                pltpu.async_copy(prop_hbm.at[pl.ds(off, EB)], pb, sem),
            ]

        def do_pass(ps, carry):
            pbase = sid * (D // 16) + ps * PC
            for j in range(PC):
                pltpu.sync_copy(xwt_hbm.at[pl.ds((pbase + j) * NP, NP)], tbls[j])

            def zacc(c, c2):
                for j in range(PC):
                    accs[j][pl.ds(c * 16, 16)] = zeros16
                return c2

            lax.fori_loop(0, NCH, zacc, 0)

            descs = fire(0, 0)
            for b in range(NBAT):
                par = b % 2
                nxt = fire(b + 1, 1 - par) if b + 1 < NBAT else None
                for d in descs:
                    d.wait()
                rb, cb, pb, _ = bufs[par]

                def chunk(k, c3):
                    for u in range(4):
                        o = k * 64 + u * 16
                        r = rb[pl.ds(o, 16)]
                        cc = cb[pl.ds(o, 16)]
                        pr = pb[pl.ds(o, 16)]
                        for j in range(PC):
                            g = plsc.load_gather(tbls[j], [r])
                            plsc.addupdate_scatter(accs[j], [cc], g * pr)
                    return c3

                lax.fori_loop(0, EB // 64, chunk, 0)
                descs = nxt
            for j in range(PC):
                pltpu.sync_copy(
                    accs[j], out_hbm.at[pl.ds((cid * D + pbase + j) * NP, NP)])
            return carry

        lax.fori_loop(0, passes, do_pass, 0)

    return body


def _make_gcn_planar(D):
    return functools.partial(
        pl.kernel,
        out_type=jax.ShapeDtypeStruct((2 * D * NP,), jnp.float32),
        mesh=plsc.VectorSubcoreMesh(core_axis_name="c", subcore_axis_name="s"),
        compiler_params=pltpu.CompilerParams(needs_layout_passes=False),
        scratch_types=[
            pltpu.VMEM((EB,), jnp.int32),
            pltpu.VMEM((EB,), jnp.int32),
            pltpu.VMEM((EB,), jnp.float32),
            pltpu.VMEM((EB,), jnp.int32),
            pltpu.VMEM((EB,), jnp.int32),
            pltpu.VMEM((EB,), jnp.float32),
        ] + [pltpu.VMEM((NP,), jnp.float32)] * 8 + [
            pltpu.SemaphoreType.DMA,
            pltpu.SemaphoreType.DMA,
        ],
    )(_gcn_planar_body(D))


_gcn_planar_256 = _make_gcn_planar(256)
_gcn_planar_128 = _make_gcn_planar(128)


def _seg_scalar_body(idx_hbm, val_hbm, out_hbm, idx_v, val_v, acc_v):
    cid = lax.axis_index("c")
    sid = lax.axis_index("s")
    wid = sid * 2 + cid
    base = wid * EPW
    pltpu.sync_copy(idx_hbm.at[pl.ds(base, EPW)], idx_v)
    pltpu.sync_copy(val_hbm.at[pl.ds(base, EPW)], val_v)
    zeros16 = jnp.zeros((16,), jnp.float32)

    def zacc(c, c2):
        acc_v[pl.ds(c * 16, 16)] = zeros16
        return c2

    lax.fori_loop(0, NCH, zacc, 0)

    def chunk(k, c2):
        o = k * 16
        plsc.addupdate_scatter(acc_v, [idx_v[pl.ds(o, 16)]], val_v[pl.ds(o, 16)])
        return c2

    lax.fori_loop(0, EPW // 16, chunk, 0)
    pltpu.sync_copy(acc_v, out_hbm.at[pl.ds(wid * NP, NP)])


_seg_scalar = functools.partial(
    pl.kernel,
    out_type=jax.ShapeDtypeStruct((NW * NP,), jnp.float32),
    mesh=plsc.VectorSubcoreMesh(core_axis_name="c", subcore_axis_name="s"),
    compiler_params=pltpu.CompilerParams(needs_layout_passes=False),
    scratch_types=[
        pltpu.VMEM((EPW,), jnp.int32),
        pltpu.VMEM((EPW,), jnp.float32),
        pltpu.VMEM((NP,), jnp.float32),
    ],
)(_seg_scalar_body)


def _seg_abs_cnt_body(idx_hbm, ev_hbm, s_hbm, cnt_hbm, idx_v, ev_v, acc1_v, acc2_v):
    cid = lax.axis_index("c")
    sid = lax.axis_index("s")
    wid = sid * 2 + cid
    base = wid * EPW
    pltpu.sync_copy(idx_hbm.at[pl.ds(base, EPW)], idx_v)
    pltpu.sync_copy(ev_hbm.at[pl.ds(base, EPW)], ev_v)
    zeros16 = jnp.zeros((16,), jnp.float32)
    ones16 = jnp.ones((16,), jnp.float32)

    def zacc(c, c2):
        acc1_v[pl.ds(c * 16, 16)] = zeros16
        acc2_v[pl.ds(c * 16, 16)] = zeros16
        return c2

    lax.fori_loop(0, NCH, zacc, 0)

    def chunk(k, c2):
        o = k * 16
        r = idx_v[pl.ds(o, 16)]
        plsc.addupdate_scatter(acc1_v, [r], jnp.abs(ev_v[pl.ds(o, 16)]))
        plsc.addupdate_scatter(acc2_v, [r], ones16)
        return c2

    lax.fori_loop(0, EPW // 16, chunk, 0)
    pltpu.sync_copy(acc1_v, s_hbm.at[pl.ds(wid * NP, NP)])
    pltpu.sync_copy(acc2_v, cnt_hbm.at[pl.ds(wid * NP, NP)])


_seg_abs_cnt = functools.partial(
    pl.kernel,
    out_type=(
        jax.ShapeDtypeStruct((NW * NP,), jnp.float32),
        jax.ShapeDtypeStruct((NW * NP,), jnp.float32),
    ),
    mesh=plsc.VectorSubcoreMesh(core_axis_name="c", subcore_axis_name="s"),
    compiler_params=pltpu.CompilerParams(needs_layout_passes=False),
    scratch_types=[
        pltpu.VMEM((EPW,), jnp.int32),
        pltpu.VMEM((EPW,), jnp.float32),
        pltpu.VMEM((NP,), jnp.float32),
        pltpu.VMEM((NP,), jnp.float32),
    ],
)(_seg_abs_cnt_body)


def _emax_body(e_hbm, out_hbm, e_v, m_v):
    cid = lax.axis_index("c")
    sid = lax.axis_index("s")
    wid = sid * 2 + cid
    base = wid * EPW
    pltpu.sync_copy(e_hbm.at[pl.ds(base, EPW)], e_v)

    def mx(k, m):
        return jnp.maximum(m, e_v[pl.ds(k * 16, 16)])

    m = lax.fori_loop(1, EPW // 16, mx, e_v[pl.ds(0, 16)])
    m_v[pl.ds(0, 16)] = m
    pltpu.sync_copy(m_v, out_hbm.at[pl.ds(wid * 16, 16)])


_emax = functools.partial(
    pl.kernel,
    out_type=jax.ShapeDtypeStruct((NW * 16,), jnp.float32),
    mesh=plsc.VectorSubcoreMesh(core_axis_name="c", subcore_axis_name="s"),
    compiler_params=pltpu.CompilerParams(needs_layout_passes=False),
    scratch_types=[
        pltpu.VMEM((EPW,), jnp.float32),
        pltpu.VMEM((16,), jnp.float32),
    ],
)(_emax_body)


def _exp_body(e_hbm, m_hbm, out_hbm, e_v, m_v, o_v):
    cid = lax.axis_index("c")
    sid = lax.axis_index("s")
    wid = sid * 2 + cid
    base = wid * EPW
    pltpu.sync_copy(e_hbm.at[pl.ds(base, EPW)], e_v)
    pltpu.sync_copy(m_hbm, m_v)
    m16 = m_v[pl.ds(0, 16)]

    def chunk(k, c2):
        o = k * 16
        o_v[pl.ds(o, 16)] = jnp.exp(e_v[pl.ds(o, 16)] - m16)
        return c2

    lax.fori_loop(0, EPW // 16, chunk, 0)
    pltpu.sync_copy(o_v, out_hbm.at[pl.ds(base, EPW)])


_exp_kernel = functools.partial(
    pl.kernel,
    out_type=jax.ShapeDtypeStruct((E,), jnp.float32),
    mesh=plsc.VectorSubcoreMesh(core_axis_name="c", subcore_axis_name="s"),
    compiler_params=pltpu.CompilerParams(needs_layout_passes=False),
    scratch_types=[
        pltpu.VMEM((EPW,), jnp.float32),
        pltpu.VMEM((16,), jnp.float32),
        pltpu.VMEM((EPW,), jnp.float32),
    ],
)(_exp_body)


def _leaky(x):
    return jnp.where(x >= 0, x, 0.1 * x)


NB = 512
EBK = 6400


def _dense1_body(x1_ref, w1_ref, b1_ref, w2_ref, lb2_ref, out_ref):
    x = w1_ref[...] * x1_ref[...] + b1_ref[...]
    x = _leaky(x)
    out_ref[...] = lax.dot_general(
        w2_ref[...], x, (((0,), (0,)), ((), ())),
        preferred_element_type=jnp.float32) + lb2_ref[...]


_dense1 = pl.pallas_call(
    _dense1_body,
    grid=(NP // NB,),
    in_specs=[
        pl.BlockSpec((1, NB), lambda i: (0, i)),
        pl.BlockSpec((FD, 1), lambda i: (0, 0)),
        pl.BlockSpec((FD, 1), lambda i: (0, 0)),
        pl.BlockSpec((FD, 2 * FD), lambda i: (0, 0)),
        pl.BlockSpec((2 * FD, 1), lambda i: (0, 0)),
    ],
    out_specs=pl.BlockSpec((2 * FD, NB), lambda i: (0, i)),
    out_shape=jax.ShapeDtypeStruct((2 * FD, NP), jnp.float32),
)


def _dense2_body(g0_ref, g1_ref, pd_ref, xw_ref, b_ref, w_ref, lb_ref, out_ref):
    x = _leaky(g0_ref[...] + g1_ref[...] + pd_ref[...] * xw_ref[...] + b_ref[...])
    out_ref[...] = lax.dot_general(
        w_ref[...], x, (((0,), (0,)), ((), ())),
        preferred_element_type=jnp.float32) + lb_ref[...]


def _make_dense2(Din, Dout):
    return pl.pallas_call(
        _dense2_body,
        grid=(NP // NB,),
        in_specs=[
            pl.BlockSpec((Din, NB), lambda i: (0, i)),
            pl.BlockSpec((Din, NB), lambda i: (0, i)),
            pl.BlockSpec((1, NB), lambda i: (0, i)),
            pl.BlockSpec((Din, NB), lambda i: (0, i)),
            pl.BlockSpec((Din, 1), lambda i: (0, 0)),
            pl.BlockSpec((Din, Dout), lambda i: (0, 0)),
            pl.BlockSpec((Dout, 1), lambda i: (0, 0)),
        ],
        out_specs=pl.BlockSpec((Dout, NB), lambda i: (0, i)),
        out_shape=jax.ShapeDtypeStruct((Dout, NP), jnp.float32),
    )


_dense2_256_128 = _make_dense2(2 * FD, FD)
_dense2_128_32 = _make_dense2(FD, 32)


def _attmlp_body(xet_ref, vn_ref, a2t_ref, a2v_ref, a2b_ref, cw_ref, out_ref):
    x = xet_ref[...]
    y = jnp.dot(a2t_ref[...], x, preferred_element_type=jnp.float32)
    y = y + a2v_ref[...] * vn_ref[...] + a2b_ref[...]
    y = _leaky(y)
    out_ref[...] = jnp.dot(cw_ref[...], y, preferred_element_type=jnp.float32)


_attmlp = pl.pallas_call(
    _attmlp_body,
    grid=(E // EBK,),
    in_specs=[
        pl.BlockSpec((16, EBK), lambda i: (0, i)),
        pl.BlockSpec((1, EBK), lambda i: (0, i)),
        pl.BlockSpec((17, 16), lambda i: (0, 0)),
        pl.BlockSpec((17, 1), lambda i: (0, 0)),
        pl.BlockSpec((17, 1), lambda i: (0, 0)),
        pl.BlockSpec((1, 17), lambda i: (0, 0)),
    ],
    out_specs=pl.BlockSpec((1, EBK), lambda i: (0, i)),
    out_shape=jax.ShapeDtypeStruct((1, E), jnp.float32),
)


def kernel(edge_index, edge_val, lin1_W, lin1_b, c2_W, c2_lb, c2_b, c3_W, c3_lb, c3_b, a1_W, a1_b, a2_W, a2_b, cW, cb, T):
    n = N
    row = edge_index[0]
    col = edge_index[1]
    ev = edge_val
    abs_val = jnp.abs(ev)

    def pad_n(v):
        return jnp.pad(v, (0, NP - n))

    sp, cp = _seg_abs_cnt(row, ev)
    s = sp.reshape(NW, NP).sum(0)[:n]
    cnt = cp.reshape(NW, NP).sum(0)[:n]
    abs_mean = s / jnp.maximum(cnt, 1.0)
    deg = s + jnp.abs(abs_mean)
    dinv = 1.0 / jnp.sqrt(jnp.maximum(deg, 1e-12))
    prop_e, vnorm = _prop_kernel(pad_n(dinv), pad_n(s), row, col, ev)
    prop_d = dinv * abs_mean * dinv

    x1 = _seg_scalar(col, prop_e).reshape(NW, NP).sum(0)[:n] + prop_d
    x1p = jnp.pad(x1, (0, NP - n))[None]
    pd_p = jnp.pad(prop_d, (0, NP - n))[None]
    xw2 = _dense1(x1p, lin1_W.T, lin1_b[:, None], c2_W, c2_lb[:, None])
    p2 = _gcn_planar_256(xw2.reshape(-1), row, col, prop_e).reshape(2, 2 * FD, NP)
    xw3 = _dense2_256_128(p2[0], p2[1], pd_p, xw2, c2_b[:, None], c3_W,
                          c3_lb[:, None])
    p3 = _gcn_planar_128(xw3.reshape(-1), row, col, prop_e).reshape(2, FD, NP)
    a1_cat = jnp.concatenate([a1_W[:FD], a1_W[FD:]], axis=1)
    g12 = _dense2_128_32(p3[0], p3[1], pd_p, xw3, c3_b[:, None], a1_cat,
                         jnp.zeros((32, 1), jnp.float32))

    g1t = g12[:16].reshape(-1)
    g2t = g12[16:].reshape(-1)
    xet = _att_pair(g1t, g2t, row, col, a1_b).reshape(16, E)
    e = _attmlp(xet, vnorm[None], a2_W[:-1].T, a2_W[-1][:, None],
                a2_b[:, None], cW.T)[0] + cb[0]
    M = jnp.max(_emax(e))
    ex = _exp_kernel(e, jnp.broadcast_to(M, (16,)))
    ssum = _seg_scalar(row, ex).reshape(NW, NP).sum(0)[:n]

    dve = _dve_kernel(pad_n(ssum), ex, row, col, ev)
    nid = jnp.arange(NP)
    dvd = jnp.where(nid < n, jnp.where(nid == 0, 1.0, 1.0 - DT), 0.0)
    dvd = dvd.astype(jnp.float32)
    spins, _, _ = _diffusion_kernel(row, col, dve, dvd)
    return spins[:n][:, None]


# speedup vs baseline: 1.0220x; 1.0093x over previous
import functools

import jax
import jax.numpy as jnp
from jax import lax
from jax.experimental import pallas as pl
from jax.experimental.pallas import tpu as pltpu
from jax.experimental.pallas import tpu_sc as plsc

N = 10000
E = 320000
FD = 128
DT = 0.1
STEPS = 100

NSC = 16
NCH = 640
NP = NCH * 16
SL = NP // NSC
EPT = E // NSC


def _diffusion_body(row_hbm, col_hbm, dve_hbm, dvd_hbm,
                    out_hbm, part_hbm, sfin_hbm,
                    row_v, col_v, dve_v, s_v, acc_v, dvd_v, rbuf_v, own_v, sem_d):
    cid = lax.axis_index("c")
    sid = lax.axis_index("s")
    ebase = sid * EPT
    pltpu.sync_copy(row_hbm.at[pl.ds(ebase, EPT)], row_v)
    pltpu.sync_copy(col_hbm.at[pl.ds(ebase, EPT)], col_v)
    pltpu.sync_copy(dve_hbm.at[pl.ds(ebase, EPT)], dve_v)
    pltpu.sync_copy(dvd_hbm, dvd_v)

    zeros16 = jnp.zeros((16,), jnp.float32)
    ones16 = jnp.ones((16,), jnp.float32)
    lanes = lax.iota(jnp.int32, 16)
    sbase = sid * SL

    def init_body(c, carry):
        s_v[pl.ds(c * 16, 16)] = ones16
        return carry

    lax.fori_loop(0, NCH, init_body, 0)

    def zacc(c, c2):
        for u in range(4):
            acc_v[pl.ds(c * 64 + u * 16, 16)] = zeros16
        return c2

    lax.fori_loop(0, NCH // 4, zacc, 0)

    def step(it, carry):
        par = lax.rem(it, 2)

        def edge(k, c2):
            for u in range(5):
                off = k * 80 + u * 16
                idx = col_v[pl.ds(off, 16)]
                g = plsc.load_gather(s_v, [idx])
                contrib = g * dve_v[pl.ds(off, 16)]
                r = row_v[pl.ds(off, 16)]
                plsc.addupdate_scatter(acc_v, [r], contrib)
            return c2

        lax.fori_loop(0, EPT // 80, edge, 0)
        pltpu.sync_copy(acc_v, part_hbm.at[par, cid, sid])
        plsc.subcore_barrier()

        descs = [
            pltpu.async_copy(
                part_hbm.at[par, cid, j, pl.ds(sbase, SL)], rbuf_v.at[j], sem_d)
            for j in range(NSC)
        ]
        for d in descs:
            d.wait()

        def red(j, c2):
            off = j * 16
            tot = dvd_v[pl.ds(sbase + off, 16)] * s_v[pl.ds(sbase + off, 16)]

            def radd(k, t):
                return t + rbuf_v[k, pl.ds(off, 16)]

            tot = lax.fori_loop(0, NSC, radd, tot)
            own_v[pl.ds(off, 16)] = tot
            return c2

        lax.fori_loop(0, SL // 16, red, 0)
        pltpu.sync_copy(own_v, sfin_hbm.at[par, cid, pl.ds(sbase, SL)])
        plsc.subcore_barrier()
        rb_d = pltpu.async_copy(sfin_hbm.at[par, cid], s_v, sem_d)
        lax.fori_loop(0, NCH // 4, zacc, 0)
        rb_d.wait()
        return carry

    lax.fori_loop(0, STEPS, step, 0)

    @pl.when((cid == 0) & (sid == 0))
    def _finalize():
        def mx(c, m):
            a = jnp.abs(s_v[pl.ds(c * 16, 16)])
            a = jnp.where(c == 0, jnp.where(lanes > 0, a, zeros16), a)
            return jnp.maximum(m, a)

        m_vec = lax.fori_loop(0, NCH, mx, zeros16)
        m = jnp.max(m_vec)
        scale = ones16 / jnp.maximum(jnp.broadcast_to(m, (16,)), 1e-12)

        def wr(c, c2):
            o = s_v[pl.ds(c * 16, 16)] * scale
            o = jnp.where(c == 0, jnp.where(lanes == 0, ones16, o), o)
            acc_v[pl.ds(c * 16, 16)] = o
            return c2

        lax.fori_loop(0, NCH, wr, 0)
        pltpu.sync_copy(acc_v, out_hbm)


_diffusion_kernel = functools.partial(
    pl.kernel,
    out_type=(
        jax.ShapeDtypeStruct((NP,), jnp.float32),
        jax.ShapeDtypeStruct((2, 2, NSC, NP), jnp.float32),
        jax.ShapeDtypeStruct((2, 2, NP), jnp.float32),
    ),
    mesh=plsc.VectorSubcoreMesh(core_axis_name="c", subcore_axis_name="s"),
    compiler_params=pltpu.CompilerParams(needs_layout_passes=False),
    scratch_types=[
        pltpu.VMEM((EPT,), jnp.int32),
        pltpu.VMEM((EPT,), jnp.int32),
        pltpu.VMEM((EPT,), jnp.float32),
        pltpu.VMEM((NP,), jnp.float32),
        pltpu.VMEM((NP,), jnp.float32),
        pltpu.VMEM((NP,), jnp.float32),
        pltpu.VMEM((NSC, SL), jnp.float32),
        pltpu.VMEM((SL,), jnp.float32),
        pltpu.SemaphoreType.DMA,
    ],
)(_diffusion_body)


NW = 32
EPW = E // NW
GB = 80


def _att_pair_body(g1t_hbm, g2t_hbm, row_hbm, col_hbm, b_hbm, out_hbm,
                   g1a, g2a, g1b, g2b, row_v, col_v, b_v, obuf_v, semA, semB):
    cid = lax.axis_index("c")
    sid = lax.axis_index("s")
    wid = sid * 2 + cid
    base = wid * EPW
    pltpu.sync_copy(row_hbm.at[pl.ds(base, EPW)], row_v)
    pltpu.sync_copy(col_hbm.at[pl.ds(base, EPW)], col_v)
    pltpu.sync_copy(b_hbm, b_v)
    tb = [(g1a, g2a, semA), (g1b, g2b, semB)]

    def fire(k, par):
        t1, t2, sem = tb[par]
        return [
            pltpu.async_copy(g1t_hbm.at[pl.ds(k * NP, NP)], t1, sem),
            pltpu.async_copy(g2t_hbm.at[pl.ds(k * NP, NP)], t2, sem),
        ]

    descs = fire(0, 0)
    for k in range(16):
        par = k % 2
        nxt = fire(k + 1, 1 - par) if k < 15 else None
        for d in descs:
            d.wait()
        t1, t2, _ = tb[par]
        bk = plsc.load_gather(b_v, [jnp.full((16,), k, jnp.int32)])

        def chunk(c, c2):
            off = c * 16
            r = row_v[pl.ds(off, 16)]
            cc = col_v[pl.ds(off, 16)]
            x = plsc.load_gather(t1, [r]) + plsc.load_gather(t2, [cc]) + bk
            obuf_v[pl.ds(off, 16)] = jnp.where(x >= 0, x, 0.1 * x)
            return c2

        lax.fori_loop(0, EPW // 16, chunk, 0)
        pltpu.sync_copy(obuf_v, out_hbm.at[pl.ds(k * E + base, EPW)])
        descs = nxt


_att_pair = functools.partial(
    pl.kernel,
    out_type=jax.ShapeDtypeStruct((16 * E,), jnp.float32),
    mesh=plsc.VectorSubcoreMesh(core_axis_name="c", subcore_axis_name="s"),
    compiler_params=pltpu.CompilerParams(needs_layout_passes=False),
    scratch_types=[
        pltpu.VMEM((NP,), jnp.float32),
        pltpu.VMEM((NP,), jnp.float32),
        pltpu.VMEM((NP,), jnp.float32),
        pltpu.VMEM((NP,), jnp.float32),
        pltpu.VMEM((EPW,), jnp.int32),
        pltpu.VMEM((EPW,), jnp.int32),
        pltpu.VMEM((16,), jnp.float32),
        pltpu.VMEM((EPW,), jnp.float32),
        pltpu.SemaphoreType.DMA,
        pltpu.SemaphoreType.DMA,
    ],
)(_att_pair_body)


def _prop_body(dinv_hbm, s_hbm, row_hbm, col_hbm, ev_hbm, prop_hbm, vnorm_hbm,
               dinv_v, s_v, row_v, col_v, ev_v, o1_v, o2_v):
    cid = lax.axis_index("c")
    sid = lax.axis_index("s")
    wid = sid * 2 + cid
    base = wid * EPW
    pltpu.sync_copy(dinv_hbm, dinv_v)
    pltpu.sync_copy(s_hbm, s_v)
    pltpu.sync_copy(row_hbm.at[pl.ds(base, EPW)], row_v)
    pltpu.sync_copy(col_hbm.at[pl.ds(base, EPW)], col_v)
    pltpu.sync_copy(ev_hbm.at[pl.ds(base, EPW)], ev_v)
    eps16 = jnp.full((16,), 1e-12, jnp.float32)

    def chunk(k, carry):
        off = k * 16
        r = row_v[pl.ds(off, 16)]
        c = col_v[pl.ds(off, 16)]
        v = ev_v[pl.ds(off, 16)]
        dr = plsc.load_gather(dinv_v, [r])
        dc = plsc.load_gather(dinv_v, [c])
        o1_v[pl.ds(off, 16)] = dr * v * dc
        sr = plsc.load_gather(s_v, [r])
        o2_v[pl.ds(off, 16)] = jnp.abs(v) / jnp.maximum(sr, eps16)
        return carry

    lax.fori_loop(0, EPW // 16, chunk, 0)
    pltpu.sync_copy(o1_v, prop_hbm.at[pl.ds(base, EPW)])
    pltpu.sync_copy(o2_v, vnorm_hbm.at[pl.ds(base, EPW)])


_prop_kernel = functools.partial(
    pl.kernel,
    out_type=(
        jax.ShapeDtypeStruct((E,), jnp.float32),
        jax.ShapeDtypeStruct((E,), jnp.float32),
    ),
    mesh=plsc.VectorSubcoreMesh(core_axis_name="c", subcore_axis_name="s"),
    compiler_params=pltpu.CompilerParams(needs_layout_passes=False),
    scratch_types=[
        pltpu.VMEM((NP,), jnp.float32),
        pltpu.VMEM((NP,), jnp.float32),
        pltpu.VMEM((EPW,), jnp.int32),
        pltpu.VMEM((EPW,), jnp.int32),
        pltpu.VMEM((EPW,), jnp.float32),
        pltpu.VMEM((EPW,), jnp.float32),
        pltpu.VMEM((EPW,), jnp.float32),
    ],
)(_prop_body)


def _dve_body(ssum_hbm, ex_hbm, row_hbm, col_hbm, ev_hbm, dve_hbm,
              ssum_v, ex_v, row_v, col_v, ev_v, o_v):
    cid = lax.axis_index("c")
    sid = lax.axis_index("s")
    wid = sid * 2 + cid
    base = wid * EPW
    pltpu.sync_copy(ssum_hbm, ssum_v)
    pltpu.sync_copy(ex_hbm.at[pl.ds(base, EPW)], ex_v)
    pltpu.sync_copy(row_hbm.at[pl.ds(base, EPW)], row_v)
    pltpu.sync_copy(col_hbm.at[pl.ds(base, EPW)], col_v)
    pltpu.sync_copy(ev_hbm.at[pl.ds(base, EPW)], ev_v)
    eps16 = jnp.full((16,), 1e-12, jnp.float32)
    zeros16 = jnp.zeros((16,), jnp.float32)
    ones16 = jnp.ones((16,), jnp.float32)
    dt16 = jnp.full((16,), DT, jnp.float32)

    def chunk(k, carry):
        off = k * 16
        r = row_v[pl.ds(off, 16)]
        c = col_v[pl.ds(off, 16)]
        sr = plsc.load_gather(ssum_v, [r])
        a = ex_v[pl.ds(off, 16)] / jnp.maximum(sr, eps16)
        pol = -jnp.sign(ev_v[pl.ds(off, 16)]) * a
        dve = dt16 * pol
        dve = jnp.where(r == 0, jnp.where(c == 0, ones16, zeros16), dve)
        o_v[pl.ds(off, 16)] = dve
        return carry

    lax.fori_loop(0, EPW // 16, chunk, 0)
    pltpu.sync_copy(o_v, dve_hbm.at[pl.ds(base, EPW)])


_dve_kernel = functools.partial(
    pl.kernel,
    out_type=jax.ShapeDtypeStruct((E,), jnp.float32),
    mesh=plsc.VectorSubcoreMesh(core_axis_name="c", subcore_axis_name="s"),
    compiler_params=pltpu.CompilerParams(needs_layout_passes=False),
    scratch_types=[
        pltpu.VMEM((NP,), jnp.float32),
        pltpu.VMEM((EPW,), jnp.float32),
        pltpu.VMEM((EPW,), jnp.int32),
        pltpu.VMEM((EPW,), jnp.int32),
        pltpu.VMEM((EPW,), jnp.float32),
        pltpu.VMEM((EPW,), jnp.float32),
    ],
)(_dve_body)


EB = 6400
PC = 4


def _gcn_planar_body(D):
    EPC = E // 2
    passes = (D // 16) // PC
    NBAT = EPC // EB

    def body(xwt_hbm, row_hbm, col_hbm, prop_hbm, out_hbm,
             rb0, cb0, pb0, rb1, cb1, pb1,
             t0, t1, t2, t3, a0, a1, a2, a3, sem0, sem1):
        cid = lax.axis_index("c")
        sid = lax.axis_index("s")
        ebase = cid * EPC
        zeros16 = jnp.zeros((16,), jnp.float32)
        tbls = [t0, t1, t2, t3]
        accs = [a0, a1, a2, a3]
        bufs = [(rb0, cb0, pb0, sem0), (rb1, cb1, pb1, sem1)]

        def fire(b, par):
            off = ebase + b * EB
            rb, cb, pb, sem = bufs[par]
            return [
                pltpu.async_copy(row_hbm.at[pl.ds(off, EB)], rb, sem),
                pltpu.async_copy(col_hbm.at[pl.ds(off, EB)], cb, sem),
                pltpu.async_copy(prop_hbm.at[pl.ds(off, EB)], pb, sem),
            ]

        def do_pass(ps, carry):
            pbase = sid * (D // 16) + ps * PC
            for j in range(PC):
                pltpu.sync_copy(xwt_hbm.at[pl.ds((pbase + j) * NP, NP)], tbls[j])

            def zacc(c, c2):
                for j in range(PC):
                    accs[j][pl.ds(c * 16, 16)] = zeros16
                return c2

            lax.fori_loop(0, NCH, zacc, 0)

            descs = fire(0, 0)
            for b in range(NBAT):
                par = b % 2
                nxt = fire(b + 1, 1 - par) if b + 1 < NBAT else None
                for d in descs:
                    d.wait()
                rb, cb, pb, _ = bufs[par]

                def chunk(k, c3):
                    for u in range(4):
                        o = k * 64 + u * 16
                        r = rb[pl.ds(o, 16)]
                        cc = cb[pl.ds(o, 16)]
                        pr = pb[pl.ds(o, 16)]
                        for j in range(PC):
                            g = plsc.load_gather(tbls[j], [r])
                            plsc.addupdate_scatter(accs[j], [cc], g * pr)
                    return c3

                lax.fori_loop(0, EB // 64, chunk, 0)
                descs = nxt
            for j in range(PC):
                pltpu.sync_copy(
                    accs[j], out_hbm.at[pl.ds((cid * D + pbase + j) * NP, NP)])
            return carry

        lax.fori_loop(0, passes, do_pass, 0)

    return body


def _make_gcn_planar(D):
    return functools.partial(
        pl.kernel,
        out_type=jax.ShapeDtypeStruct((2 * D * NP,), jnp.float32),
        mesh=plsc.VectorSubcoreMesh(core_axis_name="c", subcore_axis_name="s"),
        compiler_params=pltpu.CompilerParams(needs_layout_passes=False),
        scratch_types=[
            pltpu.VMEM((EB,), jnp.int32),
            pltpu.VMEM((EB,), jnp.int32),
            pltpu.VMEM((EB,), jnp.float32),
            pltpu.VMEM((EB,), jnp.int32),
            pltpu.VMEM((EB,), jnp.int32),
            pltpu.VMEM((EB,), jnp.float32),
        ] + [pltpu.VMEM((NP,), jnp.float32)] * 8 + [
            pltpu.SemaphoreType.DMA,
            pltpu.SemaphoreType.DMA,
        ],
    )(_gcn_planar_body(D))


_gcn_planar_256 = _make_gcn_planar(256)
_gcn_planar_128 = _make_gcn_planar(128)


def _seg_scalar_body(idx_hbm, val_hbm, out_hbm, idx_v, val_v, acc_v):
    cid = lax.axis_index("c")
    sid = lax.axis_index("s")
    wid = sid * 2 + cid
    base = wid * EPW
    pltpu.sync_copy(idx_hbm.at[pl.ds(base, EPW)], idx_v)
    pltpu.sync_copy(val_hbm.at[pl.ds(base, EPW)], val_v)
    zeros16 = jnp.zeros((16,), jnp.float32)

    def zacc(c, c2):
        acc_v[pl.ds(c * 16, 16)] = zeros16
        return c2

    lax.fori_loop(0, NCH, zacc, 0)

    def chunk(k, c2):
        o = k * 16
        plsc.addupdate_scatter(acc_v, [idx_v[pl.ds(o, 16)]], val_v[pl.ds(o, 16)])
        return c2

    lax.fori_loop(0, EPW // 16, chunk, 0)
    pltpu.sync_copy(acc_v, out_hbm.at[pl.ds(wid * NP, NP)])


_seg_scalar = functools.partial(
    pl.kernel,
    out_type=jax.ShapeDtypeStruct((NW * NP,), jnp.float32),
    mesh=plsc.VectorSubcoreMesh(core_axis_name="c", subcore_axis_name="s"),
    compiler_params=pltpu.CompilerParams(needs_layout_passes=False),
    scratch_types=[
        pltpu.VMEM((EPW,), jnp.int32),
        pltpu.VMEM((EPW,), jnp.float32),
        pltpu.VMEM((NP,), jnp.float32),
    ],
)(_seg_scalar_body)


def _seg_abs_cnt_body(idx_hbm, ev_hbm, s_hbm, cnt_hbm, idx_v, ev_v, acc1_v, acc2_v):
    cid = lax.axis_index("c")
    sid = lax.axis_index("s")
    wid = sid * 2 + cid
    base = wid * EPW
    pltpu.sync_copy(idx_hbm.at[pl.ds(base, EPW)], idx_v)
    pltpu.sync_copy(ev_hbm.at[pl.ds(base, EPW)], ev_v)
    zeros16 = jnp.zeros((16,), jnp.float32)
    ones16 = jnp.ones((16,), jnp.float32)

    def zacc(c, c2):
        acc1_v[pl.ds(c * 16, 16)] = zeros16
        acc2_v[pl.ds(c * 16, 16)] = zeros16
        return c2

    lax.fori_loop(0, NCH, zacc, 0)

    def chunk(k, c2):
        o = k * 16
        r = idx_v[pl.ds(o, 16)]
        plsc.addupdate_scatter(acc1_v, [r], jnp.abs(ev_v[pl.ds(o, 16)]))
        plsc.addupdate_scatter(acc2_v, [r], ones16)
        return c2

    lax.fori_loop(0, EPW // 16, chunk, 0)
    pltpu.sync_copy(acc1_v, s_hbm.at[pl.ds(wid * NP, NP)])
    pltpu.sync_copy(acc2_v, cnt_hbm.at[pl.ds(wid * NP, NP)])


_seg_abs_cnt = functools.partial(
    pl.kernel,
    out_type=(
        jax.ShapeDtypeStruct((NW * NP,), jnp.float32),
        jax.ShapeDtypeStruct((NW * NP,), jnp.float32),
    ),
    mesh=plsc.VectorSubcoreMesh(core_axis_name="c", subcore_axis_name="s"),
    compiler_params=pltpu.CompilerParams(needs_layout_passes=False),
    scratch_types=[
        pltpu.VMEM((EPW,), jnp.int32),
        pltpu.VMEM((EPW,), jnp.float32),
        pltpu.VMEM((NP,), jnp.float32),
        pltpu.VMEM((NP,), jnp.float32),
    ],
)(_seg_abs_cnt_body)


def _emax_body(e_hbm, out_hbm, e_v, m_v):
    cid = lax.axis_index("c")
    sid = lax.axis_index("s")
    wid = sid * 2 + cid
    base = wid * EPW
    pltpu.sync_copy(e_hbm.at[pl.ds(base, EPW)], e_v)

    def mx(k, m):
        return jnp.maximum(m, e_v[pl.ds(k * 16, 16)])

    m = lax.fori_loop(1, EPW // 16, mx, e_v[pl.ds(0, 16)])
    m_v[pl.ds(0, 16)] = m
    pltpu.sync_copy(m_v, out_hbm.at[pl.ds(wid * 16, 16)])


_emax = functools.partial(
    pl.kernel,
    out_type=jax.ShapeDtypeStruct((NW * 16,), jnp.float32),
    mesh=plsc.VectorSubcoreMesh(core_axis_name="c", subcore_axis_name="s"),
    compiler_params=pltpu.CompilerParams(needs_layout_passes=False),
    scratch_types=[
        pltpu.VMEM((EPW,), jnp.float32),
        pltpu.VMEM((16,), jnp.float32),
    ],
)(_emax_body)


def _exp_body(e_hbm, m_hbm, out_hbm, e_v, m_v, o_v):
    cid = lax.axis_index("c")
    sid = lax.axis_index("s")
    wid = sid * 2 + cid
    base = wid * EPW
    pltpu.sync_copy(e_hbm.at[pl.ds(base, EPW)], e_v)
    pltpu.sync_copy(m_hbm, m_v)
    m16 = m_v[pl.ds(0, 16)]

    def chunk(k, c2):
        o = k * 16
        o_v[pl.ds(o, 16)] = jnp.exp(e_v[pl.ds(o, 16)] - m16)
        return c2

    lax.fori_loop(0, EPW // 16, chunk, 0)
    pltpu.sync_copy(o_v, out_hbm.at[pl.ds(base, EPW)])


_exp_kernel = functools.partial(
    pl.kernel,
    out_type=jax.ShapeDtypeStruct((E,), jnp.float32),
    mesh=plsc.VectorSubcoreMesh(core_axis_name="c", subcore_axis_name="s"),
    compiler_params=pltpu.CompilerParams(needs_layout_passes=False),
    scratch_types=[
        pltpu.VMEM((EPW,), jnp.float32),
        pltpu.VMEM((16,), jnp.float32),
        pltpu.VMEM((EPW,), jnp.float32),
    ],
)(_exp_body)


def _leaky(x):
    return jnp.where(x >= 0, x, 0.1 * x)


NB = 512
EBK = 6400


def _dense1_body(x1_ref, w1_ref, b1_ref, w2_ref, lb2_ref, out_ref):
    x = w1_ref[...] * x1_ref[...] + b1_ref[...]
    x = _leaky(x)
    out_ref[...] = lax.dot_general(
        w2_ref[...], x, (((0,), (0,)), ((), ())),
        preferred_element_type=jnp.float32) + lb2_ref[...]


_dense1 = pl.pallas_call(
    _dense1_body,
    grid=(NP // NB,),
    in_specs=[
        pl.BlockSpec((1, NB), lambda i: (0, i)),
        pl.BlockSpec((FD, 1), lambda i: (0, 0)),
        pl.BlockSpec((FD, 1), lambda i: (0, 0)),
        pl.BlockSpec((FD, 2 * FD), lambda i: (0, 0)),
        pl.BlockSpec((2 * FD, 1), lambda i: (0, 0)),
    ],
    out_specs=pl.BlockSpec((2 * FD, NB), lambda i: (0, i)),
    out_shape=jax.ShapeDtypeStruct((2 * FD, NP), jnp.float32),
)


def _dense2_body(g0_ref, g1_ref, pd_ref, xw_ref, b_ref, w_ref, lb_ref, out_ref):
    x = _leaky(g0_ref[...] + g1_ref[...] + pd_ref[...] * xw_ref[...] + b_ref[...])
    out_ref[...] = lax.dot_general(
        w_ref[...], x, (((0,), (0,)), ((), ())),
        preferred_element_type=jnp.float32) + lb_ref[...]


def _make_dense2(Din, Dout):
    return pl.pallas_call(
        _dense2_body,
        grid=(NP // NB,),
        in_specs=[
            pl.BlockSpec((Din, NB), lambda i: (0, i)),
            pl.BlockSpec((Din, NB), lambda i: (0, i)),
            pl.BlockSpec((1, NB), lambda i: (0, i)),
            pl.BlockSpec((Din, NB), lambda i: (0, i)),
            pl.BlockSpec((Din, 1), lambda i: (0, 0)),
            pl.BlockSpec((Din, Dout), lambda i: (0, 0)),
            pl.BlockSpec((Dout, 1), lambda i: (0, 0)),
        ],
        out_specs=pl.BlockSpec((Dout, NB), lambda i: (0, i)),
        out_shape=jax.ShapeDtypeStruct((Dout, NP), jnp.float32),
    )


_dense2_256_128 = _make_dense2(2 * FD, FD)
_dense2_128_32 = _make_dense2(FD, 32)


def _attmlp_body(xet_ref, vn_ref, a2t_ref, a2v_ref, a2b_ref, cw_ref, out_ref):
    x = xet_ref[...]
    y = jnp.dot(a2t_ref[...], x, preferred_element_type=jnp.float32)
    y = y + a2v_ref[...] * vn_ref[...] + a2b_ref[...]
    y = _leaky(y)
    out_ref[...] = jnp.dot(cw_ref[...], y, preferred_element_type=jnp.float32)


_attmlp = pl.pallas_call(
    _attmlp_body,
    grid=(E // EBK,),
    in_specs=[
        pl.BlockSpec((16, EBK), lambda i: (0, i)),
        pl.BlockSpec((1, EBK), lambda i: (0, i)),
        pl.BlockSpec((17, 16), lambda i: (0, 0)),
        pl.BlockSpec((17, 1), lambda i: (0, 0)),
        pl.BlockSpec((17, 1), lambda i: (0, 0)),
        pl.BlockSpec((1, 17), lambda i: (0, 0)),
    ],
    out_specs=pl.BlockSpec((1, EBK), lambda i: (0, i)),
    out_shape=jax.ShapeDtypeStruct((1, E), jnp.float32),
)


def kernel(edge_index, edge_val, lin1_W, lin1_b, c2_W, c2_lb, c2_b, c3_W, c3_lb, c3_b, a1_W, a1_b, a2_W, a2_b, cW, cb, T):
    n = N
    row = edge_index[0]
    col = edge_index[1]
    ev = edge_val
    abs_val = jnp.abs(ev)

    def pad_n(v):
        return jnp.pad(v, (0, NP - n))

    sp, cp = _seg_abs_cnt(row, ev)
    s = sp.reshape(NW, NP).sum(0)[:n]
    cnt = cp.reshape(NW, NP).sum(0)[:n]
    abs_mean = s / jnp.maximum(cnt, 1.0)
    deg = s + jnp.abs(abs_mean)
    dinv = 1.0 / jnp.sqrt(jnp.maximum(deg, 1e-12))
    prop_e, vnorm = _prop_kernel(pad_n(dinv), pad_n(s), row, col, ev)
    prop_d = dinv * abs_mean * dinv

    x1 = _seg_scalar(col, prop_e).reshape(NW, NP).sum(0)[:n] + prop_d
    x1p = jnp.pad(x1, (0, NP - n))[None]
    pd_p = jnp.pad(prop_d, (0, NP - n))[None]
    xw2 = _dense1(x1p, lin1_W.T, lin1_b[:, None], c2_W, c2_lb[:, None])
    p2 = _gcn_planar_256(xw2.reshape(-1), row, col, prop_e).reshape(2, 2 * FD, NP)
    xw3 = _dense2_256_128(p2[0], p2[1], pd_p, xw2, c2_b[:, None], c3_W,
                          c3_lb[:, None])
    p3 = _gcn_planar_128(xw3.reshape(-1), row, col, prop_e).reshape(2, FD, NP)
    a1_cat = jnp.concatenate([a1_W[:FD], a1_W[FD:]], axis=1)
    g12 = _dense2_128_32(p3[0], p3[1], pd_p, xw3, c3_b[:, None], a1_cat,
                         jnp.zeros((32, 1), jnp.float32))

    g1t = g12[:16].reshape(-1)
    g2t = g12[16:].reshape(-1)
    xet = _att_pair(g1t, g2t, row, col, a1_b).reshape(16, E)
    e = _attmlp(xet, vnorm[None], a2_W[:-1].T, a2_W[-1][:, None],
                a2_b[:, None], cW.T)[0] + cb[0]
    M = jnp.max(_emax(e))
    ex = _exp_kernel(e, jnp.broadcast_to(M, (16,)))
    ssum = _seg_scalar(row, ex).reshape(NW, NP).sum(0)[:n]

    dve = _dve_kernel(pad_n(ssum), ex, row, col, ev)
    nid = jnp.arange(NP)
    dvd = jnp.where(nid < n, jnp.where(nid == 0, 1.0, 1.0 - DT), 0.0)
    dvd = dvd.astype(jnp.float32)
    spins, _, _ = _diffusion_kernel(row, col, dve, dvd)
    return spins[:n][:, None]
